# baseline (speedup 1.0000x reference)
import functools
import math

import jax
import jax.numpy as jnp
from jax import lax
from jax.experimental import pallas as pl
from jax.experimental.pallas import tpu as pltpu
from jax.experimental.pallas import tpu_sc as plsc

_N = 10000
_DH = 128
_HD = 64
_E = 320000
_G = 64
_C = 10
_BN_EPS = 1e-5
_INV_BN = 1.0 / math.sqrt(1.0 + _BN_EPS)

_NC = 2
_NS = 16
_CHUNK = 128
_CPW = 160
_EPAD = _NS * _CPW * _CHUNK
_NPAD = 10112
_RPT = _NPAD // _NS

_RB = 1000


@functools.partial(
    pl.kernel,
    out_type=jax.ShapeDtypeStruct((_NC, _NPAD, _HD), jnp.float32),
    mesh=plsc.VectorSubcoreMesh(core_axis_name="c", subcore_axis_name="s",
                                num_cores=_NC, num_subcores=_NS),
    compiler_params=pltpu.CompilerParams(use_tc_tiling_on_sc=False),
    scratch_types=[
        pltpu.VMEM((_CPW, _CHUNK), jnp.int32),
        pltpu.VMEM((_CPW, _CHUNK), jnp.int32),
        pltpu.VMEM((2, 2 * _CHUNK, _HD), jnp.float32),
        pltpu.VMEM_SHARED((_NPAD, _HD), jnp.float32),
        [pltpu.SemaphoreType.DMA] * 2,
        [pltpu.SemaphoreType.DMA] * 2,
    ],
)
def _sc_agg(h2_hbm, src_hbm, dst_hbm, zeros_hbm, out_hbm,
            src_v, dst_v, rows, acc_sh, gsem, ssem):
    c = lax.axis_index("c")
    s = lax.axis_index("s")
    h_view = h2_hbm.at[c]
    nb = _CPW // 2

    def fire_gathers(t, p):
        for u in range(2):
            pltpu.async_copy(h_view.at[src_v.at[t * 2 + u]],
                             rows.at[p, pl.ds(u * _CHUNK, _CHUNK)], gsem[p])

    def fire_scatters(t, p):
        for u in range(2):
            pltpu.async_copy(rows.at[p, pl.ds(u * _CHUNK, _CHUNK)],
                             acc_sh.at[dst_v.at[t * 2 + u]], ssem[p],
                             add=True)

    def wait_gathers(p):
        pltpu.make_async_copy(h_view.at[pl.ds(0, 2 * _CHUNK)], rows.at[p],
                              gsem[p]).wait()

    def wait_scatters(p):
        pltpu.make_async_copy(rows.at[p], acc_sh.at[pl.ds(0, 2 * _CHUNK)],
                              ssem[p]).wait()

    pltpu.sync_copy(src_hbm.at[s], src_v)
    pltpu.sync_copy(dst_hbm.at[s], dst_v)
    fire_gathers(0, 0)
    pltpu.sync_copy(zeros_hbm, acc_sh.at[pl.ds(s * _RPT, _RPT)])
    plsc.subcore_barrier()

    def body(j, carry):
        for p in range(2):
            t = j * 2 + p

            wait_gathers(p)
            fire_scatters(t, p)

            @pl.when(t + 1 < nb)
            def _():
                @pl.when(t >= 1)
                def _():
                    wait_scatters(1 - p)

                fire_gathers(t + 1, 1 - p)

        return carry

    lax.fori_loop(0, nb // 2, body, 0)
    wait_scatters(0)
    wait_scatters(1)
    plsc.subcore_barrier()
    pltpu.sync_copy(acc_sh.at[pl.ds(s * _RPT, _RPT)],
                    out_hbm.at[c, pl.ds(s * _RPT, _RPT)])


def _mlp_body(h_ref, p_ref, sc_ref, w1_ref, b1_ref, w2_ref, b2_ref,
              gm_ref, bt_ref, out_ref):
    h = jnp.concatenate([h_ref[0], h_ref[1]], axis=-1)
    agg = jnp.concatenate([p_ref[0], p_ref[1]], axis=-1)
    z = h * sc_ref[...] + agg
    a = jnp.dot(z, w1_ref[...], preferred_element_type=jnp.float32)
    a = jnp.maximum(a + b1_ref[...], 0.0)
    a = jnp.dot(a, w2_ref[...], preferred_element_type=jnp.float32)
    a = jnp.maximum(a + b2_ref[...], 0.0)
    y = a * gm_ref[...] + bt_ref[...]
    out_ref[0] = y[:, :_HD]
    out_ref[1] = y[:, _HD:]


def _tc_mlp(h2, parts, scale_row, w1, b1r, w2, b2r, gmr, btr):
    fix = lambda i: (0, 0)
    return pl.pallas_call(
        _mlp_body,
        grid=(_N // _RB,),
        in_specs=[
            pl.BlockSpec((_NC, _RB, _HD), lambda i: (0, i, 0)),
            pl.BlockSpec((_NC, _RB, _HD), lambda i: (0, i, 0)),
            pl.BlockSpec((1, _DH), fix),
            pl.BlockSpec((_DH, _DH), fix),
            pl.BlockSpec((1, _DH), fix),
            pl.BlockSpec((_DH, _DH), fix),
            pl.BlockSpec((1, _DH), fix),
            pl.BlockSpec((1, _DH), fix),
            pl.BlockSpec((1, _DH), fix),
        ],
        out_specs=pl.BlockSpec((_NC, _RB, _HD), lambda i: (0, i, 0)),
        out_shape=jax.ShapeDtypeStruct((_NC, _N, _HD), jnp.float32),
    )(h2, parts, scale_row, w1, b1r, w2, b2r, gmr, btr)


def _head_body(h_ref, b_ref, w1_ref, b1_ref, w2_ref, b2_ref, out_ref,
               acc_ref, cnt_ref):
    step = pl.program_id(0)

    @pl.when(step == 0)
    def _():
        acc_ref[...] = jnp.zeros_like(acc_ref)
        cnt_ref[...] = jnp.zeros_like(cnt_ref)

    h = jnp.concatenate([h_ref[0], h_ref[1]], axis=-1)
    bids = b_ref[0, 0, :]
    onehot = (bids[:, None] == lax.broadcasted_iota(jnp.int32, (_RB, _G), 1))
    onehot = onehot.astype(jnp.float32)
    dn = (((0,), (0,)), ((), ()))
    acc_ref[...] += lax.dot_general(onehot, h, dn,
                                    preferred_element_type=jnp.float32)
    cnt_ref[...] += lax.dot_general(
        onehot, jnp.ones((_RB, _DH), jnp.float32), dn,
        preferred_element_type=jnp.float32)

    @pl.when(step == pl.num_programs(0) - 1)
    def _():
        pooled = acc_ref[...] / jnp.maximum(cnt_ref[...], 1.0)
        o = jnp.dot(pooled, w1_ref[...], preferred_element_type=jnp.float32)
        o = jnp.maximum(o + b1_ref[...], 0.0)
        o = jnp.dot(o, w2_ref[...], preferred_element_type=jnp.float32)
        o = o + b2_ref[...]
        m = jnp.max(o, axis=-1, keepdims=True)
        lse = jnp.log(jnp.sum(jnp.exp(o - m), axis=-1, keepdims=True)) + m
        out_ref[...] = o - lse


def _tc_head(h2, batch3d, w1, b1r, w2, b2r):
    fix = lambda i: (0, 0)
    return pl.pallas_call(
        _head_body,
        grid=(_N // _RB,),
        in_specs=[
            pl.BlockSpec((_NC, _RB, _HD), lambda i: (0, i, 0)),
            pl.BlockSpec((1, 1, _RB), lambda i: (i, 0, 0)),
            pl.BlockSpec((_DH, _DH), fix),
            pl.BlockSpec((1, _DH), fix),
            pl.BlockSpec((_DH, _C), fix),
            pl.BlockSpec((1, _C), fix),
        ],
        out_specs=pl.BlockSpec((_G, _C), fix),
        out_shape=jax.ShapeDtypeStruct((_G, _C), jnp.float32),
        scratch_shapes=[
            pltpu.VMEM((_G, _DH), jnp.float32),
            pltpu.VMEM((_G, _DH), jnp.float32),
        ],
    )(h2, batch3d, w1, b1r, w2, b2r)


def kernel(x, edge_index, batch, params):
    src = edge_index[0].astype(jnp.int32)
    dst = edge_index[1].astype(jnp.int32)
    pad = _EPAD - _E
    src_p = jnp.concatenate([src, jnp.zeros((pad,), jnp.int32)])
    src_p = src_p.reshape(_NS, _CPW, _CHUNK)
    dst_p = jnp.concatenate([dst, jnp.full((pad,), _N, jnp.int32)])
    dst_p = dst_p.reshape(_NS, _CPW, _CHUNK)
    zeros_rows = jnp.zeros((_RPT, _HD), jnp.float32)
    batch3d = batch.astype(jnp.int32).reshape(_N // _RB, 1, _RB)

    h2 = jnp.stack([x[:, :_HD], x[:, _HD:]])
    for p in params["convs"]:
        parts = _sc_agg(h2, src_p, dst_p, zeros_rows)
        scale_row = (1.0 + p["eps"]) * jnp.ones((1, _DH), jnp.float32)
        h2 = _tc_mlp(h2, parts, scale_row,
                     p["W1"], p["b1"].reshape(1, _DH),
                     p["W2"], p["b2"].reshape(1, _DH),
                     (p["gamma"] * _INV_BN).reshape(1, _DH),
                     p["beta"].reshape(1, _DH))

    return _tc_head(h2, batch3d,
                    params["lin1_W"], params["lin1_b"].reshape(1, _DH),
                    params["lin2_W"], params["lin2_b"].reshape(1, _C))


# speedup vs baseline: 1.5886x; 1.5886x over previous
import functools
import math

import jax
import jax.numpy as jnp
from jax import lax
from jax.experimental import pallas as pl
from jax.experimental.pallas import tpu as pltpu
from jax.experimental.pallas import tpu_sc as plsc

_N = 10000
_DH = 128
_HD = 64
_E = 320000
_G = 64
_C = 10
_BN_EPS = 1e-5
_INV_BN = 1.0 / math.sqrt(1.0 + _BN_EPS)

_NC = 2
_NS = 16
_CHUNK = 128
_CPW = 160
_EPAD = _NS * _CPW * _CHUNK
_NPAD = 10112
_RPT = _NPAD // _NS

_RB = 1000


@functools.partial(
    pl.kernel,
    out_type=jax.ShapeDtypeStruct((_NC, _NPAD, _HD), jnp.bfloat16),
    mesh=plsc.VectorSubcoreMesh(core_axis_name="c", subcore_axis_name="s",
                                num_cores=_NC, num_subcores=_NS),
    compiler_params=pltpu.CompilerParams(use_tc_tiling_on_sc=False),
    scratch_types=[
        pltpu.VMEM((_CPW, _CHUNK), jnp.int32),
        pltpu.VMEM((_CPW, _CHUNK), jnp.int32),
        pltpu.VMEM((_CHUNK, _HD), jnp.bfloat16),
        pltpu.VMEM((_CHUNK, _HD), jnp.bfloat16),
        pltpu.VMEM_SHARED((_NPAD, _HD), jnp.bfloat16),
        pltpu.SemaphoreType.DMA,
        pltpu.SemaphoreType.DMA,
    ],
)
def _sc_agg(h2_hbm, src_hbm, dst_hbm, zeros_hbm, out_hbm,
            src_v, dst_v, rows0, rows1, acc_sh, sem0, sem1):
    c = lax.axis_index("c")
    s = lax.axis_index("s")
    h_view = h2_hbm.at[c]

    pltpu.sync_copy(src_hbm.at[s], src_v)
    pltpu.sync_copy(dst_hbm.at[s], dst_v)
    pltpu.async_copy(h_view.at[src_v.at[0]], rows0, sem0)
    pltpu.async_copy(h_view.at[src_v.at[1]], rows1, sem1)
    pltpu.sync_copy(zeros_hbm, acc_sh.at[pl.ds(s * _RPT, _RPT)])
    plsc.subcore_barrier()

    def body(j, carry):
        i0 = j * 2
        pltpu.make_async_copy(h_view.at[src_v.at[i0]], rows0, sem0).wait()
        pltpu.sync_copy(rows0, acc_sh.at[dst_v.at[i0]], add=True)

        @pl.when(i0 + 2 < _CPW)
        def _():
            pltpu.async_copy(h_view.at[src_v.at[i0 + 2]], rows0, sem0)

        i1 = i0 + 1
        pltpu.make_async_copy(h_view.at[src_v.at[i1]], rows1, sem1).wait()
        pltpu.sync_copy(rows1, acc_sh.at[dst_v.at[i1]], add=True)

        @pl.when(i1 + 2 < _CPW)
        def _():
            pltpu.async_copy(h_view.at[src_v.at[i1 + 2]], rows1, sem1)

        return carry

    lax.fori_loop(0, _CPW // 2, body, 0)
    plsc.subcore_barrier()
    pltpu.sync_copy(acc_sh.at[pl.ds(s * _RPT, _RPT)],
                    out_hbm.at[c, pl.ds(s * _RPT, _RPT)])


def _mlp_body(h_ref, p_ref, sc_ref, w1_ref, b1_ref, w2_ref, b2_ref,
              gm_ref, bt_ref, out_ref, outb_ref):
    h = jnp.concatenate([h_ref[0], h_ref[1]], axis=-1)
    agg = jnp.concatenate([p_ref[0], p_ref[1]], axis=-1)
    z = h * sc_ref[...] + agg.astype(jnp.float32)
    a = jnp.dot(z, w1_ref[...], preferred_element_type=jnp.float32)
    a = jnp.maximum(a + b1_ref[...], 0.0)
    a = jnp.dot(a, w2_ref[...], preferred_element_type=jnp.float32)
    a = jnp.maximum(a + b2_ref[...], 0.0)
    y = a * gm_ref[...] + bt_ref[...]
    out_ref[0] = y[:, :_HD]
    out_ref[1] = y[:, _HD:]
    yb = y.astype(jnp.bfloat16)
    outb_ref[0] = yb[:, :_HD]
    outb_ref[1] = yb[:, _HD:]


def _tc_mlp(h2, parts, scale_row, w1, b1r, w2, b2r, gmr, btr):
    fix = lambda i: (0, 0)
    return pl.pallas_call(
        _mlp_body,
        grid=(_N // _RB,),
        in_specs=[
            pl.BlockSpec((_NC, _RB, _HD), lambda i: (0, i, 0)),
            pl.BlockSpec((_NC, _RB, _HD), lambda i: (0, i, 0)),
            pl.BlockSpec((1, _DH), fix),
            pl.BlockSpec((_DH, _DH), fix),
            pl.BlockSpec((1, _DH), fix),
            pl.BlockSpec((_DH, _DH), fix),
            pl.BlockSpec((1, _DH), fix),
            pl.BlockSpec((1, _DH), fix),
            pl.BlockSpec((1, _DH), fix),
        ],
        out_specs=(
            pl.BlockSpec((_NC, _RB, _HD), lambda i: (0, i, 0)),
            pl.BlockSpec((_NC, _RB, _HD), lambda i: (0, i, 0)),
        ),
        out_shape=(
            jax.ShapeDtypeStruct((_NC, _N, _HD), jnp.float32),
            jax.ShapeDtypeStruct((_NC, _N, _HD), jnp.bfloat16),
        ),
    )(h2, parts, scale_row, w1, b1r, w2, b2r, gmr, btr)


def _head_body(h_ref, b_ref, w1_ref, b1_ref, w2_ref, b2_ref, out_ref,
               acc_ref, cnt_ref):
    step = pl.program_id(0)

    @pl.when(step == 0)
    def _():
        acc_ref[...] = jnp.zeros_like(acc_ref)
        cnt_ref[...] = jnp.zeros_like(cnt_ref)

    h = jnp.concatenate([h_ref[0], h_ref[1]], axis=-1)
    bids = b_ref[0, 0, :]
    onehot = (bids[:, None] == lax.broadcasted_iota(jnp.int32, (_RB, _G), 1))
    onehot = onehot.astype(jnp.float32)
    dn = (((0,), (0,)), ((), ()))
    acc_ref[...] += lax.dot_general(onehot, h, dn,
                                    preferred_element_type=jnp.float32)
    cnt_ref[...] += lax.dot_general(
        onehot, jnp.ones((_RB, _DH), jnp.float32), dn,
        preferred_element_type=jnp.float32)

    @pl.when(step == pl.num_programs(0) - 1)
    def _():
        pooled = acc_ref[...] / jnp.maximum(cnt_ref[...], 1.0)
        o = jnp.dot(pooled, w1_ref[...], preferred_element_type=jnp.float32)
        o = jnp.maximum(o + b1_ref[...], 0.0)
        o = jnp.dot(o, w2_ref[...], preferred_element_type=jnp.float32)
        o = o + b2_ref[...]
        m = jnp.max(o, axis=-1, keepdims=True)
        lse = jnp.log(jnp.sum(jnp.exp(o - m), axis=-1, keepdims=True)) + m
        out_ref[...] = o - lse


def _tc_head(h2, batch3d, w1, b1r, w2, b2r):
    fix = lambda i: (0, 0)
    return pl.pallas_call(
        _head_body,
        grid=(_N // _RB,),
        in_specs=[
            pl.BlockSpec((_NC, _RB, _HD), lambda i: (0, i, 0)),
            pl.BlockSpec((1, 1, _RB), lambda i: (i, 0, 0)),
            pl.BlockSpec((_DH, _DH), fix),
            pl.BlockSpec((1, _DH), fix),
            pl.BlockSpec((_DH, _C), fix),
            pl.BlockSpec((1, _C), fix),
        ],
        out_specs=pl.BlockSpec((_G, _C), fix),
        out_shape=jax.ShapeDtypeStruct((_G, _C), jnp.float32),
        scratch_shapes=[
            pltpu.VMEM((_G, _DH), jnp.float32),
            pltpu.VMEM((_G, _DH), jnp.float32),
        ],
    )(h2, batch3d, w1, b1r, w2, b2r)


def kernel(x, edge_index, batch, params):
    src = edge_index[0].astype(jnp.int32)
    dst = edge_index[1].astype(jnp.int32)
    pad = _EPAD - _E
    src_p = jnp.concatenate([src, jnp.zeros((pad,), jnp.int32)])
    src_p = src_p.reshape(_NS, _CPW, _CHUNK)
    dst_p = jnp.concatenate([dst, jnp.full((pad,), _N, jnp.int32)])
    dst_p = dst_p.reshape(_NS, _CPW, _CHUNK)
    zeros_rows = jnp.zeros((_RPT, _HD), jnp.bfloat16)
    batch3d = batch.astype(jnp.int32).reshape(_N // _RB, 1, _RB)

    h2 = jnp.stack([x[:, :_HD], x[:, _HD:]])
    h2b = h2.astype(jnp.bfloat16)
    for p in params["convs"]:
        parts = _sc_agg(h2b, src_p, dst_p, zeros_rows)
        scale_row = (1.0 + p["eps"]) * jnp.ones((1, _DH), jnp.float32)
        h2, h2b = _tc_mlp(h2, parts, scale_row,
                          p["W1"], p["b1"].reshape(1, _DH),
                          p["W2"], p["b2"].reshape(1, _DH),
                          (p["gamma"] * _INV_BN).reshape(1, _DH),
                          p["beta"].reshape(1, _DH))

    return _tc_head(h2, batch3d,
                    params["lin1_W"], params["lin1_b"].reshape(1, _DH),
                    params["lin2_W"], params["lin2_b"].reshape(1, _C))
